# baseline (speedup 1.0000x reference)
import functools

import jax
import jax.numpy as jnp
from jax import lax
from jax.experimental import pallas as pl
from jax.experimental.pallas import tpu as pltpu
from jax.experimental.pallas import tpu_sc as plsc

N_PROT = 10000
N_DRUG = 1024
TOT = N_PROT + N_DRUG
NP = 10240
NT = 11264
E_PP = 320000
E_DP = 32768
NC, NS, B = 2, 16, 128
NBW_PP1 = 160
TOTB = NS * NBW_PP1
EP_PAD = TOTB * B
NBW_CNT = TOTB // (NC * NS)
NBW_DP = 8

_MESH = plsc.VectorSubcoreMesh(core_axis_name="c", subcore_axis_name="s")


def _zero_fill(ref, nrows, d):
    def row(i, _):
        for k in range(d // 16):
            ref[i, pl.ds(16 * k, 16)] = jnp.zeros((16,), jnp.float32)
        return 0
    lax.fori_loop(0, nrows, row, 0)


def _make_sc_aggregate(nr, d, nbw, single_core):
    rows_pt = nr // NS
    out_shape = (nr, d) if single_core else (NC, nr, d)

    @functools.partial(
        pl.kernel,
        out_type=jax.ShapeDtypeStruct(out_shape, jnp.float32),
        mesh=_MESH,
        scratch_types=[
            pltpu.VMEM((nbw, B), jnp.int32),
            pltpu.VMEM((nbw, B), jnp.int32),
            [pltpu.VMEM((B, d), jnp.float32) for _ in range(4)],
            pltpu.VMEM((64, d), jnp.float32),
            pltpu.VMEM_SHARED((nr, d), jnp.float32),
            [pltpu.SemaphoreType.DMA for _ in range(4)],
            pltpu.SemaphoreType.DMA,
        ],
        compiler_params=pltpu.CompilerParams(use_tc_tiling_on_sc=False),
    )
    def agg(table_hbm, src_hbm, dst_hbm, out_hbm, sidx, didx, rows,
            zbuf, acc, gsem, zsem):
        c = lax.axis_index("c")
        s = lax.axis_index("s")

        def body():
            _zero_fill(zbuf, 64, d)

            nz = rows_pt // 64
            def zacc(i, _):
                pltpu.async_copy(
                    zbuf, acc.at[pl.ds(s * rows_pt + i * 64, 64)], zsem)
                return 0
            lax.fori_loop(0, nz, zacc, 0)

            def zdrain(i, _):
                pltpu.make_async_copy(
                    zbuf, acc.at[pl.ds(s * rows_pt, 64)], zsem).wait()
                return 0
            lax.fori_loop(0, nz, zdrain, 0)
            plsc.subcore_barrier()

            base = (s if single_core else c * NS + s) * nbw
            pltpu.sync_copy(src_hbm.at[pl.ds(base, nbw)], sidx)
            pltpu.sync_copy(dst_hbm.at[pl.ds(base, nbw)], didx)

            def gather(j, k):
                return pltpu.make_async_copy(table_hbm.at[sidx.at[j]],
                                             rows[k], gsem[k])

            nq = nbw // 4
            gather(0, 0).start()
            gather(1, 1).start()
            gather(2, 2).start()

            def step(q, _):
                for k in range(4):
                    j = 4 * q + k
                    kp = (k + 3) % 4
                    gather(j, k).wait()
                    pltpu.sync_copy(rows[k], acc.at[didx.at[j]], add=True)
                    if k == 0:
                        gather(j + 3, kp).start()
                    else:
                        @pl.when(q < nq - 1)
                        def _():
                            gather(j + 3, kp).start()
                return 0
            lax.fori_loop(0, nq, step, 0)
            plsc.subcore_barrier()

            dst = (out_hbm if single_core else out_hbm.at[c])
            pltpu.sync_copy(acc.at[pl.ds(s * rows_pt, rows_pt)],
                            dst.at[pl.ds(s * rows_pt, rows_pt)])

        if single_core:
            @pl.when(c == 0)
            def _():
                body()
        else:
            body()

    return agg


@functools.partial(
    pl.kernel,
    out_type=[jax.ShapeDtypeStruct((NC * NP,), jnp.float32),
              jax.ShapeDtypeStruct((NC * NT,), jnp.float32)],
    mesh=_MESH,
    scratch_types=[
        pltpu.VMEM((NBW_CNT, B), jnp.int32),
        pltpu.VMEM((NBW_DP, B), jnp.int32),
        pltpu.VMEM((B,), jnp.float32),
        pltpu.VMEM((NT // NS,), jnp.float32),
        pltpu.VMEM_SHARED((NP,), jnp.float32),
        pltpu.VMEM_SHARED((NT,), jnp.float32),
        pltpu.SemaphoreType.DMA,
    ],
    compiler_params=pltpu.CompilerParams(use_tc_tiling_on_sc=False),
)
def _sc_counts(col_hbm, dpd_hbm, outp_hbm, outt_hbm,
               cidx, didx, ones, zbuf, accp, acct, csem):
    c = lax.axis_index("c")
    s = lax.axis_index("s")
    pp_pt = NP // NS
    tt_pt = NT // NS
    def fill(i, _):
        zbuf[pl.ds(i * 16, 16)] = jnp.zeros((16,), jnp.float32)
        return 0
    lax.fori_loop(0, tt_pt // 16, fill, 0)
    for k in range(B // 16):
        ones[pl.ds(16 * k, 16)] = jnp.ones((16,), jnp.float32)
    pltpu.sync_copy(zbuf.at[pl.ds(0, pp_pt)], accp.at[pl.ds(s * pp_pt, pp_pt)])
    pltpu.sync_copy(zbuf, acct.at[pl.ds(s * tt_pt, tt_pt)])
    plsc.subcore_barrier()

    w = c * NS + s
    pltpu.sync_copy(col_hbm.at[pl.ds(w * NBW_CNT, NBW_CNT)], cidx)
    pltpu.sync_copy(dpd_hbm.at[pl.ds(w * NBW_DP, NBW_DP)], didx)

    def cbatch(j, _):
        pltpu.sync_copy(ones, accp.at[cidx.at[j]], add=True)
        return 0
    lax.fori_loop(0, NBW_CNT, cbatch, 0)

    def dbatch(j, _):
        pltpu.sync_copy(ones, acct.at[didx.at[j]], add=True)
        return 0
    lax.fori_loop(0, NBW_DP, dbatch, 0)
    plsc.subcore_barrier()

    pltpu.sync_copy(accp.at[pl.ds(s * pp_pt, pp_pt)], zbuf.at[pl.ds(0, pp_pt)])
    pltpu.sync_copy(zbuf.at[pl.ds(0, pp_pt)],
                    outp_hbm.at[pl.ds(c * NP + s * pp_pt, pp_pt)])
    pltpu.sync_copy(acct.at[pl.ds(s * tt_pt, tt_pt)], zbuf)
    pltpu.sync_copy(zbuf, outt_hbm.at[pl.ds(c * NT + s * tt_pt, tt_pt)])


def _tc_a(x_ref, w1_ref, cnt_ref, s1_ref, dinv_ref):
    cnt = cnt_ref[...]
    deg = lax.dot_general(cnt, jnp.ones((2, 1), jnp.float32),
                          (((0,), (0,)), ((), ()))) + 1.0
    valid = lax.broadcasted_iota(jnp.int32, (NP, 1), 0) < N_PROT
    dinv = jnp.where(valid, lax.rsqrt(deg), 0.0)
    dinv_ref[...] = dinv
    xw = jnp.dot(x_ref[...], w1_ref[...], preferred_element_type=jnp.float32)
    s1_ref[...] = jnp.zeros((NP, 32), jnp.float32)
    s1_ref[0:N_PROT, :] = xw * dinv[0:N_PROT, :]


def _tc_b(agg_ref, s1_ref, dinv_ref, b1_ref, w2_ref, s2_ref):
    dinv = dinv_ref[...]
    h1 = jax.nn.relu(dinv * (agg_ref[...] + s1_ref[...]) + b1_ref[...])
    s2_ref[...] = jnp.dot(h1, w2_ref[...],
                          preferred_element_type=jnp.float32) * dinv


def _tc_c(agg_ref, s2_ref, dinv_ref, b2_ref, h2p_ref):
    dinv = dinv_ref[...]
    valid = lax.broadcasted_iota(jnp.int32, (NP, 1), 0) < N_PROT
    h2 = jnp.where(valid,
                   dinv * (agg_ref[...] + s2_ref[...]) + b2_ref[...],
                   0.0)
    h2p_ref[...] = jnp.zeros((NT, 16), jnp.float32)
    h2p_ref[0:NP, :] = h2


def _tc_d(ssum_ref, cnt_ref, wh_ref, xd_ref, emb_ref, wf1_ref, bf1_ref,
          wf2_ref, bf2_ref, out_ref):
    ssum = ssum_ref[0][N_PROT:TOT, :] + ssum_ref[1][N_PROT:TOT, :]
    cnt = lax.dot_general(cnt_ref[...], jnp.ones((2, 1), jnp.float32),
                          (((0,), (0,)), ((), ())))
    cnt = lax.slice(cnt, (N_PROT, 0), (TOT, 1))
    aggr = ssum / jnp.maximum(cnt, 1.0)
    prot_out = jnp.dot(aggr, wh_ref[...], preferred_element_type=jnp.float32)
    xd = jnp.dot(xd_ref[...], emb_ref[...], preferred_element_type=jnp.float32)
    g = (jnp.dot(jax.nn.relu(xd), wf1_ref[0:48, :],
                 preferred_element_type=jnp.float32)
         + jnp.dot(jax.nn.relu(prot_out), wf1_ref[48:64, :],
                   preferred_element_type=jnp.float32)
         + bf1_ref[...])
    h = jax.nn.relu(g)
    f = jnp.dot(h, wf2_ref[...], preferred_element_type=jnp.float32) \
        + bf2_ref[...]
    out_ref[...] = jax.nn.sigmoid(f)


def _tc_call(fn, out_shapes):
    return pl.pallas_call(fn, out_shape=out_shapes)


def kernel(x_prot, pp_edge_index, dp_edge_index, dp_range_list, x_drug,
           W1, b1, W2, b2, Wh, embed, Wf1, bf1, Wf2, bf2):
    f32 = jnp.float32
    pp2 = jnp.pad(pp_edge_index.astype(jnp.int32),
                  ((0, 0), (0, EP_PAD - E_PP)), constant_values=NP - 1)
    src_pp = pp2[0].reshape(TOTB, B)
    col_pp = pp2[1].reshape(TOTB, B)
    src_dp = dp_edge_index[0].astype(jnp.int32).reshape(NC * NS * NBW_DP, B)
    dst_dp = dp_edge_index[1].astype(jnp.int32).reshape(NC * NS * NBW_DP, B)
    b1r, b2r = b1.reshape(1, -1), b2.reshape(1, -1)
    bf1r, bf2r = bf1.reshape(1, -1), bf2.reshape(1, -1)

    cnt_pp, cnt_dp = _sc_counts(col_pp, dst_dp)
    cnt_pp = cnt_pp.reshape(NC, NP)
    cnt_dp = cnt_dp.reshape(NC, NT)

    s1, dinv = _tc_call(_tc_a, [jax.ShapeDtypeStruct((NP, 32), f32),
                                jax.ShapeDtypeStruct((NP, 1), f32)])(
        x_prot, W1, cnt_pp)

    agg1 = _make_sc_aggregate(NP, 32, NBW_PP1, True)(s1, src_pp, col_pp)

    s2 = _tc_call(_tc_b, jax.ShapeDtypeStruct((NP, 16), f32))(
        agg1, s1, dinv, b1r, W2)

    agg2 = _make_sc_aggregate(NP, 16, NBW_PP1, True)(s2, src_pp, col_pp)

    h2p = _tc_call(_tc_c, jax.ShapeDtypeStruct((NT, 16), f32))(
        agg2, s2, dinv, b2r)

    ssum = _make_sc_aggregate(NT, 16, NBW_DP, False)(h2p, src_dp, dst_dp)

    out = _tc_call(_tc_d, jax.ShapeDtypeStruct((N_DRUG, N_DRUG), f32))(
        ssum, cnt_dp, Wh, x_drug, embed, Wf1, bf1r, Wf2, bf2r)
    return out.reshape(-1)


# speedup vs baseline: 1.1886x; 1.1886x over previous
import functools

import jax
import jax.numpy as jnp
from jax import lax
from jax.experimental import pallas as pl
from jax.experimental.pallas import tpu as pltpu
from jax.experimental.pallas import tpu_sc as plsc

N_PROT = 10000
N_DRUG = 1024
TOT = N_PROT + N_DRUG
NP = 10240
NT = 11264
E_PP = 320000
E_DP = 32768
NC, NS, B = 2, 16, 128
NB0, NB1 = 136, 24
TOTB = NS * (NB0 + NB1)
EP_PAD = TOTB * B
NBW_CNT = TOTB // (NC * NS)
NBW_DP = 8

_MESH = plsc.VectorSubcoreMesh(core_axis_name="c", subcore_axis_name="s")


def _zero_fill(ref, nrows, d):
    def row(i, _):
        for k in range(d // 16):
            ref[i, pl.ds(16 * k, 16)] = jnp.zeros((16,), jnp.float32)
        return 0
    lax.fori_loop(0, nrows, row, 0)


def _make_sc_aggregate(nr, d, nbw0, nbw1):
    rows_pt = nr // NS
    nbw_max = max(nbw0, nbw1)

    @functools.partial(
        pl.kernel,
        out_type=jax.ShapeDtypeStruct((NC, nr, d), jnp.float32),
        mesh=_MESH,
        scratch_types=[
            pltpu.VMEM((nbw_max, B), jnp.int32),
            pltpu.VMEM((nbw_max, B), jnp.int32),
            [pltpu.VMEM((B, d), jnp.float32) for _ in range(4)],
            pltpu.VMEM((64, d), jnp.float32),
            pltpu.VMEM_SHARED((nr, d), jnp.float32),
            [pltpu.SemaphoreType.DMA for _ in range(4)],
            pltpu.SemaphoreType.DMA,
        ],
        compiler_params=pltpu.CompilerParams(use_tc_tiling_on_sc=False),
    )
    def agg(table_hbm, src_hbm, dst_hbm, out_hbm, sidx, didx, rows,
            zbuf, acc, gsem, zsem):
        c = lax.axis_index("c")
        s = lax.axis_index("s")
        _zero_fill(zbuf, 64, d)

        nz = rows_pt // 64
        def zacc(i, _):
            pltpu.async_copy(
                zbuf, acc.at[pl.ds(s * rows_pt + i * 64, 64)], zsem)
            return 0
        lax.fori_loop(0, nz, zacc, 0)

        def zdrain(i, _):
            pltpu.make_async_copy(
                zbuf, acc.at[pl.ds(s * rows_pt, 64)], zsem).wait()
            return 0
        lax.fori_loop(0, nz, zdrain, 0)
        plsc.subcore_barrier()

        if nbw0 == nbw1:
            nbw = nbw0
            base = (c * NS + s) * nbw0
            pltpu.sync_copy(src_hbm.at[pl.ds(base, nbw0)], sidx)
            pltpu.sync_copy(dst_hbm.at[pl.ds(base, nbw0)], didx)
        else:
            nbw = jnp.where(c == 0, nbw0, nbw1)

            @pl.when(c == 0)
            def _():
                pltpu.sync_copy(src_hbm.at[pl.ds(s * nbw0, nbw0)],
                                sidx.at[pl.ds(0, nbw0)])
                pltpu.sync_copy(dst_hbm.at[pl.ds(s * nbw0, nbw0)],
                                didx.at[pl.ds(0, nbw0)])

            @pl.when(c == 1)
            def _():
                pltpu.sync_copy(src_hbm.at[pl.ds(NS * nbw0 + s * nbw1, nbw1)],
                                sidx.at[pl.ds(0, nbw1)])
                pltpu.sync_copy(dst_hbm.at[pl.ds(NS * nbw0 + s * nbw1, nbw1)],
                                didx.at[pl.ds(0, nbw1)])

        def gather(j, k):
            return pltpu.make_async_copy(table_hbm.at[sidx.at[j]],
                                         rows[k], gsem[k])

        nq = nbw // 4
        gather(0, 0).start()
        gather(1, 1).start()
        gather(2, 2).start()

        def step(q, _):
            for k in range(4):
                j = 4 * q + k
                kp = (k + 3) % 4
                gather(j, k).wait()
                pltpu.sync_copy(rows[k], acc.at[didx.at[j]], add=True)
                if k == 0:
                    gather(j + 3, kp).start()
                else:
                    @pl.when(q < nq - 1)
                    def _():
                        gather(j + 3, kp).start()
            return 0
        lax.fori_loop(0, nq, step, 0)
        plsc.subcore_barrier()

        pltpu.sync_copy(acc.at[pl.ds(s * rows_pt, rows_pt)],
                        out_hbm.at[c, pl.ds(s * rows_pt, rows_pt)])

    return agg


@functools.partial(
    pl.kernel,
    out_type=[jax.ShapeDtypeStruct((NC * NP,), jnp.float32),
              jax.ShapeDtypeStruct((NC * NT,), jnp.float32)],
    mesh=_MESH,
    scratch_types=[
        pltpu.VMEM((NBW_CNT, B), jnp.int32),
        pltpu.VMEM((NBW_DP, B), jnp.int32),
        pltpu.VMEM((B,), jnp.float32),
        pltpu.VMEM((NT // NS,), jnp.float32),
        pltpu.VMEM_SHARED((NP,), jnp.float32),
        pltpu.VMEM_SHARED((NT,), jnp.float32),
        pltpu.SemaphoreType.DMA,
    ],
    compiler_params=pltpu.CompilerParams(use_tc_tiling_on_sc=False),
)
def _sc_counts(col_hbm, dpd_hbm, outp_hbm, outt_hbm,
               cidx, didx, ones, zbuf, accp, acct, csem):
    c = lax.axis_index("c")
    s = lax.axis_index("s")
    pp_pt = NP // NS
    tt_pt = NT // NS
    def fill(i, _):
        zbuf[pl.ds(i * 16, 16)] = jnp.zeros((16,), jnp.float32)
        return 0
    lax.fori_loop(0, tt_pt // 16, fill, 0)
    for k in range(B // 16):
        ones[pl.ds(16 * k, 16)] = jnp.ones((16,), jnp.float32)
    pltpu.sync_copy(zbuf.at[pl.ds(0, pp_pt)], accp.at[pl.ds(s * pp_pt, pp_pt)])
    pltpu.sync_copy(zbuf, acct.at[pl.ds(s * tt_pt, tt_pt)])
    plsc.subcore_barrier()

    w = c * NS + s
    pltpu.sync_copy(col_hbm.at[pl.ds(w * NBW_CNT, NBW_CNT)], cidx)
    pltpu.sync_copy(dpd_hbm.at[pl.ds(w * NBW_DP, NBW_DP)], didx)

    def cbatch(j, _):
        pltpu.sync_copy(ones, accp.at[cidx.at[j]], add=True)
        return 0
    lax.fori_loop(0, NBW_CNT, cbatch, 0)

    def dbatch(j, _):
        pltpu.sync_copy(ones, acct.at[didx.at[j]], add=True)
        return 0
    lax.fori_loop(0, NBW_DP, dbatch, 0)
    plsc.subcore_barrier()

    pltpu.sync_copy(accp.at[pl.ds(s * pp_pt, pp_pt)], zbuf.at[pl.ds(0, pp_pt)])
    pltpu.sync_copy(zbuf.at[pl.ds(0, pp_pt)],
                    outp_hbm.at[pl.ds(c * NP + s * pp_pt, pp_pt)])
    pltpu.sync_copy(acct.at[pl.ds(s * tt_pt, tt_pt)], zbuf)
    pltpu.sync_copy(zbuf, outt_hbm.at[pl.ds(c * NT + s * tt_pt, tt_pt)])


def _tc_a(x_ref, w1_ref, cnt_ref, s1_ref, dinv_ref):
    cnt = cnt_ref[...]
    deg = lax.dot_general(cnt, jnp.ones((2, 1), jnp.float32),
                          (((0,), (0,)), ((), ()))) + 1.0
    valid = lax.broadcasted_iota(jnp.int32, (NP, 1), 0) < N_PROT
    dinv = jnp.where(valid, lax.rsqrt(deg), 0.0)
    dinv_ref[...] = dinv
    xw = jnp.dot(x_ref[...], w1_ref[...], preferred_element_type=jnp.float32)
    s1_ref[...] = jnp.zeros((NP, 32), jnp.float32)
    s1_ref[0:N_PROT, :] = xw * dinv[0:N_PROT, :]


def _tc_b(agg_ref, s1_ref, dinv_ref, b1_ref, w2_ref, s2_ref):
    dinv = dinv_ref[...]
    h1 = jax.nn.relu(dinv * (agg_ref[0] + agg_ref[1] + s1_ref[...])
                     + b1_ref[...])
    s2_ref[...] = jnp.dot(h1, w2_ref[...],
                          preferred_element_type=jnp.float32) * dinv


def _tc_c(agg_ref, s2_ref, dinv_ref, b2_ref, h2p_ref):
    dinv = dinv_ref[...]
    valid = lax.broadcasted_iota(jnp.int32, (NP, 1), 0) < N_PROT
    h2 = jnp.where(valid,
                   dinv * (agg_ref[0] + agg_ref[1] + s2_ref[...])
                   + b2_ref[...], 0.0)
    h2p_ref[...] = jnp.zeros((NT, 16), jnp.float32)
    h2p_ref[0:NP, :] = h2


def _tc_d(ssum_ref, cnt_ref, wh_ref, xd_ref, emb_ref, wf1_ref, bf1_ref,
          wf2_ref, bf2_ref, out_ref):
    ssum = ssum_ref[0][N_PROT:TOT, :] + ssum_ref[1][N_PROT:TOT, :]
    cnt = lax.dot_general(cnt_ref[...], jnp.ones((2, 1), jnp.float32),
                          (((0,), (0,)), ((), ())))
    cnt = lax.slice(cnt, (N_PROT, 0), (TOT, 1))
    aggr = ssum / jnp.maximum(cnt, 1.0)
    prot_out = jnp.dot(aggr, wh_ref[...], preferred_element_type=jnp.float32)
    xd = jnp.dot(xd_ref[...], emb_ref[...], preferred_element_type=jnp.float32)
    g = (jnp.dot(jax.nn.relu(xd), wf1_ref[0:48, :],
                 preferred_element_type=jnp.float32)
         + jnp.dot(jax.nn.relu(prot_out), wf1_ref[48:64, :],
                   preferred_element_type=jnp.float32)
         + bf1_ref[...])
    h = jax.nn.relu(g)
    f = jnp.dot(h, wf2_ref[...], preferred_element_type=jnp.float32) \
        + bf2_ref[...]
    out_ref[...] = jax.nn.sigmoid(f)


def _tc_call(fn, out_shapes):
    return pl.pallas_call(fn, out_shape=out_shapes)


def kernel(x_prot, pp_edge_index, dp_edge_index, dp_range_list, x_drug,
           W1, b1, W2, b2, Wh, embed, Wf1, bf1, Wf2, bf2):
    f32 = jnp.float32
    pp2 = jnp.pad(pp_edge_index.astype(jnp.int32),
                  ((0, 0), (0, EP_PAD - E_PP)), constant_values=NP - 1)
    src_pp = pp2[0].reshape(TOTB, B)
    col_pp = pp2[1].reshape(TOTB, B)
    src_dp = dp_edge_index[0].astype(jnp.int32).reshape(NC * NS * NBW_DP, B)
    dst_dp = dp_edge_index[1].astype(jnp.int32).reshape(NC * NS * NBW_DP, B)
    b1r, b2r = b1.reshape(1, -1), b2.reshape(1, -1)
    bf1r, bf2r = bf1.reshape(1, -1), bf2.reshape(1, -1)

    cnt_pp, cnt_dp = _sc_counts(col_pp, dst_dp)
    cnt_pp = cnt_pp.reshape(NC, NP)
    cnt_dp = cnt_dp.reshape(NC, NT)

    s1, dinv = _tc_call(_tc_a, [jax.ShapeDtypeStruct((NP, 32), f32),
                                jax.ShapeDtypeStruct((NP, 1), f32)])(
        x_prot, W1, cnt_pp)

    agg1 = _make_sc_aggregate(NP, 32, NB0, NB1)(s1, src_pp, col_pp)

    s2 = _tc_call(_tc_b, jax.ShapeDtypeStruct((NP, 16), f32))(
        agg1, s1, dinv, b1r, W2)

    agg2 = _make_sc_aggregate(NP, 16, NB0, NB1)(s2, src_pp, col_pp)

    h2p = _tc_call(_tc_c, jax.ShapeDtypeStruct((NT, 16), f32))(
        agg2, s2, dinv, b2r)

    ssum = _make_sc_aggregate(NT, 16, NBW_DP, NBW_DP)(h2p, src_dp, dst_dp)

    out = _tc_call(_tc_d, jax.ShapeDtypeStruct((N_DRUG, N_DRUG), f32))(
        ssum, cnt_dp, Wh, x_drug, embed, Wf1, bf1r, Wf2, bf2r)
    return out.reshape(-1)


# speedup vs baseline: 1.2036x; 1.0126x over previous
import functools

import jax
import jax.numpy as jnp
from jax import lax
from jax.experimental import pallas as pl
from jax.experimental.pallas import tpu as pltpu
from jax.experimental.pallas import tpu_sc as plsc

N_PROT = 10000
N_DRUG = 1024
TOT = N_PROT + N_DRUG
NP = 10240
NT = 11264
E_PP = 320000
E_DP = 32768
NC, NS, B = 2, 16, 128
NB0, NB1 = 144, 16
TOTB = NS * (NB0 + NB1)
EP_PAD = TOTB * B
NBW_CNT = TOTB // (NC * NS)
NBW_DP = 8

_MESH = plsc.VectorSubcoreMesh(core_axis_name="c", subcore_axis_name="s")


def _zero_fill(ref, nrows, d):
    def row(i, _):
        for k in range(d // 16):
            ref[i, pl.ds(16 * k, 16)] = jnp.zeros((16,), jnp.float32)
        return 0
    lax.fori_loop(0, nrows, row, 0)


def _make_sc_aggregate(nr, d, nbw0, nbw1):
    rows_pt = nr // NS
    nbw_max = max(nbw0, nbw1)

    @functools.partial(
        pl.kernel,
        out_type=jax.ShapeDtypeStruct((NC, nr, d), jnp.float32),
        mesh=_MESH,
        scratch_types=[
            pltpu.VMEM((nbw_max, B), jnp.int32),
            pltpu.VMEM((nbw_max, B), jnp.int32),
            [pltpu.VMEM((B, d), jnp.float32) for _ in range(4)],
            pltpu.VMEM((64, d), jnp.float32),
            pltpu.VMEM_SHARED((nr, d), jnp.float32),
            [pltpu.SemaphoreType.DMA for _ in range(4)],
            pltpu.SemaphoreType.DMA,
        ],
        compiler_params=pltpu.CompilerParams(use_tc_tiling_on_sc=False),
    )
    def agg(table_hbm, src_hbm, dst_hbm, out_hbm, sidx, didx, rows,
            zbuf, acc, gsem, zsem):
        c = lax.axis_index("c")
        s = lax.axis_index("s")
        _zero_fill(zbuf, 64, d)

        nz = rows_pt // 64
        def zacc(i, _):
            pltpu.async_copy(
                zbuf, acc.at[pl.ds(s * rows_pt + i * 64, 64)], zsem)
            return 0
        lax.fori_loop(0, nz, zacc, 0)

        def zdrain(i, _):
            pltpu.make_async_copy(
                zbuf, acc.at[pl.ds(s * rows_pt, 64)], zsem).wait()
            return 0
        lax.fori_loop(0, nz, zdrain, 0)
        plsc.subcore_barrier()

        if nbw0 == nbw1:
            nbw = nbw0
            base = (c * NS + s) * nbw0
            pltpu.sync_copy(src_hbm.at[pl.ds(base, nbw0)], sidx)
            pltpu.sync_copy(dst_hbm.at[pl.ds(base, nbw0)], didx)
        else:
            nbw = jnp.where(c == 0, nbw0, nbw1)

            @pl.when(c == 0)
            def _():
                pltpu.sync_copy(src_hbm.at[pl.ds(s * nbw0, nbw0)],
                                sidx.at[pl.ds(0, nbw0)])
                pltpu.sync_copy(dst_hbm.at[pl.ds(s * nbw0, nbw0)],
                                didx.at[pl.ds(0, nbw0)])

            @pl.when(c == 1)
            def _():
                pltpu.sync_copy(src_hbm.at[pl.ds(NS * nbw0 + s * nbw1, nbw1)],
                                sidx.at[pl.ds(0, nbw1)])
                pltpu.sync_copy(dst_hbm.at[pl.ds(NS * nbw0 + s * nbw1, nbw1)],
                                didx.at[pl.ds(0, nbw1)])

        def gather(j, k):
            return pltpu.make_async_copy(table_hbm.at[sidx.at[j]],
                                         rows[k], gsem[k])

        nq = nbw // 4
        gather(0, 0).start()
        gather(1, 1).start()
        gather(2, 2).start()

        def step(q, _):
            for k in range(4):
                j = 4 * q + k
                kp = (k + 3) % 4
                gather(j, k).wait()
                pltpu.sync_copy(rows[k], acc.at[didx.at[j]], add=True)
                if k == 0:
                    gather(j + 3, kp).start()
                else:
                    @pl.when(q < nq - 1)
                    def _():
                        gather(j + 3, kp).start()
            return 0
        lax.fori_loop(0, nq, step, 0)
        plsc.subcore_barrier()

        pltpu.sync_copy(acc.at[pl.ds(s * rows_pt, rows_pt)],
                        out_hbm.at[c, pl.ds(s * rows_pt, rows_pt)])

    return agg


@functools.partial(
    pl.kernel,
    out_type=[jax.ShapeDtypeStruct((NC * NP,), jnp.float32),
              jax.ShapeDtypeStruct((NC * NT,), jnp.float32)],
    mesh=_MESH,
    scratch_types=[
        pltpu.VMEM((NBW_CNT, B), jnp.int32),
        pltpu.VMEM((NBW_DP, B), jnp.int32),
        pltpu.VMEM((B,), jnp.float32),
        pltpu.VMEM((NT // NS,), jnp.float32),
        pltpu.VMEM_SHARED((NP,), jnp.float32),
        pltpu.VMEM_SHARED((NT,), jnp.float32),
        pltpu.SemaphoreType.DMA,
    ],
    compiler_params=pltpu.CompilerParams(use_tc_tiling_on_sc=False),
)
def _sc_counts(col_hbm, dpd_hbm, outp_hbm, outt_hbm,
               cidx, didx, ones, zbuf, accp, acct, csem):
    c = lax.axis_index("c")
    s = lax.axis_index("s")
    pp_pt = NP // NS
    tt_pt = NT // NS
    def fill(i, _):
        zbuf[pl.ds(i * 16, 16)] = jnp.zeros((16,), jnp.float32)
        return 0
    lax.fori_loop(0, tt_pt // 16, fill, 0)
    for k in range(B // 16):
        ones[pl.ds(16 * k, 16)] = jnp.ones((16,), jnp.float32)
    pltpu.sync_copy(zbuf.at[pl.ds(0, pp_pt)], accp.at[pl.ds(s * pp_pt, pp_pt)])
    pltpu.sync_copy(zbuf, acct.at[pl.ds(s * tt_pt, tt_pt)])
    plsc.subcore_barrier()

    w = c * NS + s
    pltpu.sync_copy(col_hbm.at[pl.ds(w * NBW_CNT, NBW_CNT)], cidx)
    pltpu.sync_copy(dpd_hbm.at[pl.ds(w * NBW_DP, NBW_DP)], didx)

    def cbatch(j, _):
        pltpu.sync_copy(ones, accp.at[cidx.at[j]], add=True)
        return 0
    lax.fori_loop(0, NBW_CNT, cbatch, 0)

    def dbatch(j, _):
        pltpu.sync_copy(ones, acct.at[didx.at[j]], add=True)
        return 0
    lax.fori_loop(0, NBW_DP, dbatch, 0)
    plsc.subcore_barrier()

    pltpu.sync_copy(accp.at[pl.ds(s * pp_pt, pp_pt)], zbuf.at[pl.ds(0, pp_pt)])
    pltpu.sync_copy(zbuf.at[pl.ds(0, pp_pt)],
                    outp_hbm.at[pl.ds(c * NP + s * pp_pt, pp_pt)])
    pltpu.sync_copy(acct.at[pl.ds(s * tt_pt, tt_pt)], zbuf)
    pltpu.sync_copy(zbuf, outt_hbm.at[pl.ds(c * NT + s * tt_pt, tt_pt)])


def _tc_a(x_ref, w1_ref, cnt_ref, s1_ref, dinv_ref):
    cnt = cnt_ref[...]
    deg = lax.dot_general(cnt, jnp.ones((2, 1), jnp.float32),
                          (((0,), (0,)), ((), ()))) + 1.0
    valid = lax.broadcasted_iota(jnp.int32, (NP, 1), 0) < N_PROT
    dinv = jnp.where(valid, lax.rsqrt(deg), 0.0)
    dinv_ref[...] = dinv
    xw = jnp.dot(x_ref[...], w1_ref[...], preferred_element_type=jnp.float32)
    s1_ref[...] = jnp.zeros((NP, 32), jnp.float32)
    s1_ref[0:N_PROT, :] = xw * dinv[0:N_PROT, :]


def _tc_b(agg_ref, s1_ref, dinv_ref, b1_ref, w2_ref, s2_ref):
    dinv = dinv_ref[...]
    h1 = jax.nn.relu(dinv * (agg_ref[0] + agg_ref[1] + s1_ref[...])
                     + b1_ref[...])
    s2_ref[...] = jnp.dot(h1, w2_ref[...],
                          preferred_element_type=jnp.float32) * dinv


def _tc_c(agg_ref, s2_ref, dinv_ref, b2_ref, h2p_ref):
    dinv = dinv_ref[...]
    valid = lax.broadcasted_iota(jnp.int32, (NP, 1), 0) < N_PROT
    h2 = jnp.where(valid,
                   dinv * (agg_ref[0] + agg_ref[1] + s2_ref[...])
                   + b2_ref[...], 0.0)
    h2p_ref[...] = jnp.zeros((NT, 16), jnp.float32)
    h2p_ref[0:NP, :] = h2


def _tc_d(ssum_ref, cnt_ref, wh_ref, xd_ref, emb_ref, wf1_ref, bf1_ref,
          wf2_ref, bf2_ref, out_ref):
    ssum = ssum_ref[0][N_PROT:TOT, :] + ssum_ref[1][N_PROT:TOT, :]
    cnt = lax.dot_general(cnt_ref[...], jnp.ones((2, 1), jnp.float32),
                          (((0,), (0,)), ((), ())))
    cnt = lax.slice(cnt, (N_PROT, 0), (TOT, 1))
    aggr = ssum / jnp.maximum(cnt, 1.0)
    prot_out = jnp.dot(aggr, wh_ref[...], preferred_element_type=jnp.float32)
    xd = jnp.dot(xd_ref[...], emb_ref[...], preferred_element_type=jnp.float32)
    g = (jnp.dot(jax.nn.relu(xd), wf1_ref[0:48, :],
                 preferred_element_type=jnp.float32)
         + jnp.dot(jax.nn.relu(prot_out), wf1_ref[48:64, :],
                   preferred_element_type=jnp.float32)
         + bf1_ref[...])
    h = jax.nn.relu(g)
    f = jnp.dot(h, wf2_ref[...], preferred_element_type=jnp.float32) \
        + bf2_ref[...]
    out_ref[...] = jax.nn.sigmoid(f)


def _tc_call(fn, out_shapes):
    return pl.pallas_call(fn, out_shape=out_shapes)


def kernel(x_prot, pp_edge_index, dp_edge_index, dp_range_list, x_drug,
           W1, b1, W2, b2, Wh, embed, Wf1, bf1, Wf2, bf2):
    f32 = jnp.float32
    pp2 = jnp.pad(pp_edge_index.astype(jnp.int32),
                  ((0, 0), (0, EP_PAD - E_PP)), constant_values=NP - 1)
    src_pp = pp2[0].reshape(TOTB, B)
    col_pp = pp2[1].reshape(TOTB, B)
    src_dp = dp_edge_index[0].astype(jnp.int32).reshape(NC * NS * NBW_DP, B)
    dst_dp = dp_edge_index[1].astype(jnp.int32).reshape(NC * NS * NBW_DP, B)
    b1r, b2r = b1.reshape(1, -1), b2.reshape(1, -1)
    bf1r, bf2r = bf1.reshape(1, -1), bf2.reshape(1, -1)

    cnt_pp, cnt_dp = _sc_counts(col_pp, dst_dp)
    cnt_pp = cnt_pp.reshape(NC, NP)
    cnt_dp = cnt_dp.reshape(NC, NT)

    s1, dinv = _tc_call(_tc_a, [jax.ShapeDtypeStruct((NP, 32), f32),
                                jax.ShapeDtypeStruct((NP, 1), f32)])(
        x_prot, W1, cnt_pp)

    agg1 = _make_sc_aggregate(NP, 32, NB0, NB1)(s1, src_pp, col_pp)

    s2 = _tc_call(_tc_b, jax.ShapeDtypeStruct((NP, 16), f32))(
        agg1, s1, dinv, b1r, W2)

    agg2 = _make_sc_aggregate(NP, 16, NB0, NB1)(s2, src_pp, col_pp)

    h2p = _tc_call(_tc_c, jax.ShapeDtypeStruct((NT, 16), f32))(
        agg2, s2, dinv, b2r)

    ssum = _make_sc_aggregate(NT, 16, NBW_DP, NBW_DP)(h2p, src_dp, dst_dp)

    out = _tc_call(_tc_d, jax.ShapeDtypeStruct((N_DRUG, N_DRUG), f32))(
        ssum, cnt_dp, Wh, x_drug, embed, Wf1, bf1r, Wf2, bf2r)
    return out.reshape(-1)


# speedup vs baseline: 1.2206x; 1.0141x over previous
import functools

import jax
import jax.numpy as jnp
from jax import lax
from jax.experimental import pallas as pl
from jax.experimental.pallas import tpu as pltpu
from jax.experimental.pallas import tpu_sc as plsc

N_PROT = 10000
N_DRUG = 1024
TOT = N_PROT + N_DRUG
NP = 10240
NT = 11264
E_PP = 320000
E_DP = 32768
NC, NS, B = 2, 16, 128
NB0, NB1 = 144, 16
TOTB = NS * (NB0 + NB1)
EP_PAD = TOTB * B
NBW_CNT = TOTB // (NC * NS)
NBW_DP = 8

_MESH = plsc.VectorSubcoreMesh(core_axis_name="c", subcore_axis_name="s")


def _zero_fill(ref, nrows, d):
    def row(i, _):
        for k in range(d // 16):
            ref[i, pl.ds(16 * k, 16)] = jnp.zeros((16,), jnp.float32)
        return 0
    lax.fori_loop(0, nrows, row, 0)


def _make_sc_aggregate(nr, d, nbw0, nbw1):
    rows_pt = nr // NS
    nbw_max = max(nbw0, nbw1)

    @functools.partial(
        pl.kernel,
        out_type=jax.ShapeDtypeStruct((NC, nr, d), jnp.float32),
        mesh=_MESH,
        scratch_types=[
            pltpu.VMEM((nbw_max, B), jnp.int32),
            pltpu.VMEM((nbw_max, B), jnp.int32),
            [pltpu.VMEM((B, d), jnp.float32) for _ in range(4)],
            pltpu.VMEM((64, d), jnp.float32),
            pltpu.VMEM_SHARED((nr, d), jnp.float32),
            [pltpu.SemaphoreType.DMA for _ in range(4)],
            [pltpu.SemaphoreType.DMA for _ in range(4)],
            pltpu.SemaphoreType.DMA,
        ],
        compiler_params=pltpu.CompilerParams(use_tc_tiling_on_sc=False),
    )
    def agg(table_hbm, src_hbm, dst_hbm, out_hbm, sidx, didx, rows,
            zbuf, acc, gsem, ssem, zsem):
        c = lax.axis_index("c")
        s = lax.axis_index("s")
        _zero_fill(zbuf, 64, d)

        nz = rows_pt // 64
        def zacc(i, _):
            pltpu.async_copy(
                zbuf, acc.at[pl.ds(s * rows_pt + i * 64, 64)], zsem)
            return 0
        lax.fori_loop(0, nz, zacc, 0)

        def zdrain(i, _):
            pltpu.make_async_copy(
                zbuf, acc.at[pl.ds(s * rows_pt, 64)], zsem).wait()
            return 0
        lax.fori_loop(0, nz, zdrain, 0)
        plsc.subcore_barrier()

        if nbw0 == nbw1:
            nbw = nbw0
            base = (c * NS + s) * nbw0
            pltpu.sync_copy(src_hbm.at[pl.ds(base, nbw0)], sidx)
            pltpu.sync_copy(dst_hbm.at[pl.ds(base, nbw0)], didx)
        else:
            nbw = jnp.where(c == 0, nbw0, nbw1)

            @pl.when(c == 0)
            def _():
                pltpu.sync_copy(src_hbm.at[pl.ds(s * nbw0, nbw0)],
                                sidx.at[pl.ds(0, nbw0)])
                pltpu.sync_copy(dst_hbm.at[pl.ds(s * nbw0, nbw0)],
                                didx.at[pl.ds(0, nbw0)])

            @pl.when(c == 1)
            def _():
                pltpu.sync_copy(src_hbm.at[pl.ds(NS * nbw0 + s * nbw1, nbw1)],
                                sidx.at[pl.ds(0, nbw1)])
                pltpu.sync_copy(dst_hbm.at[pl.ds(NS * nbw0 + s * nbw1, nbw1)],
                                didx.at[pl.ds(0, nbw1)])

        def gather(j, k):
            return pltpu.make_async_copy(table_hbm.at[sidx.at[j]],
                                         rows[k], gsem[k])

        def sdrain(k):
            pltpu.make_async_copy(rows[k], acc.at[didx.at[0]],
                                  ssem[k]).wait()

        nq = nbw // 4
        gather(0, 0).start()
        gather(1, 1).start()
        gather(2, 2).start()

        def step(q, _):
            for k in range(4):
                j = 4 * q + k
                kp = (k + 3) % 4
                gather(j, k).wait()
                pltpu.async_copy(rows[k], acc.at[didx.at[j]], ssem[k],
                                 add=True)
                if k == 0:
                    @pl.when(q > 0)
                    def _():
                        sdrain(kp)
                    gather(j + 3, kp).start()
                else:
                    sdrain(kp)

                    @pl.when(q < nq - 1)
                    def _():
                        gather(j + 3, kp).start()
            return 0
        lax.fori_loop(0, nq, step, 0)
        sdrain(3)
        plsc.subcore_barrier()

        pltpu.sync_copy(acc.at[pl.ds(s * rows_pt, rows_pt)],
                        out_hbm.at[c, pl.ds(s * rows_pt, rows_pt)])

    return agg


@functools.partial(
    pl.kernel,
    out_type=[jax.ShapeDtypeStruct((NC * NP,), jnp.float32),
              jax.ShapeDtypeStruct((NC * NT,), jnp.float32)],
    mesh=_MESH,
    scratch_types=[
        pltpu.VMEM((NBW_CNT, B), jnp.int32),
        pltpu.VMEM((NBW_DP, B), jnp.int32),
        pltpu.VMEM((B,), jnp.float32),
        pltpu.VMEM((NT // NS,), jnp.float32),
        pltpu.VMEM_SHARED((NP,), jnp.float32),
        pltpu.VMEM_SHARED((NT,), jnp.float32),
        pltpu.SemaphoreType.DMA,
    ],
    compiler_params=pltpu.CompilerParams(use_tc_tiling_on_sc=False),
)
def _sc_counts(col_hbm, dpd_hbm, outp_hbm, outt_hbm,
               cidx, didx, ones, zbuf, accp, acct, csem):
    c = lax.axis_index("c")
    s = lax.axis_index("s")
    pp_pt = NP // NS
    tt_pt = NT // NS
    def fill(i, _):
        zbuf[pl.ds(i * 16, 16)] = jnp.zeros((16,), jnp.float32)
        return 0
    lax.fori_loop(0, tt_pt // 16, fill, 0)
    for k in range(B // 16):
        ones[pl.ds(16 * k, 16)] = jnp.ones((16,), jnp.float32)
    pltpu.sync_copy(zbuf.at[pl.ds(0, pp_pt)], accp.at[pl.ds(s * pp_pt, pp_pt)])
    pltpu.sync_copy(zbuf, acct.at[pl.ds(s * tt_pt, tt_pt)])
    plsc.subcore_barrier()

    w = c * NS + s
    pltpu.sync_copy(col_hbm.at[pl.ds(w * NBW_CNT, NBW_CNT)], cidx)
    pltpu.sync_copy(dpd_hbm.at[pl.ds(w * NBW_DP, NBW_DP)], didx)

    def cbatch(j, _):
        pltpu.sync_copy(ones, accp.at[cidx.at[j]], add=True)
        return 0
    lax.fori_loop(0, NBW_CNT, cbatch, 0)

    def dbatch(j, _):
        pltpu.sync_copy(ones, acct.at[didx.at[j]], add=True)
        return 0
    lax.fori_loop(0, NBW_DP, dbatch, 0)
    plsc.subcore_barrier()

    pltpu.sync_copy(accp.at[pl.ds(s * pp_pt, pp_pt)], zbuf.at[pl.ds(0, pp_pt)])
    pltpu.sync_copy(zbuf.at[pl.ds(0, pp_pt)],
                    outp_hbm.at[pl.ds(c * NP + s * pp_pt, pp_pt)])
    pltpu.sync_copy(acct.at[pl.ds(s * tt_pt, tt_pt)], zbuf)
    pltpu.sync_copy(zbuf, outt_hbm.at[pl.ds(c * NT + s * tt_pt, tt_pt)])


def _tc_a(x_ref, w1_ref, cnt_ref, s1_ref, dinv_ref):
    cnt = cnt_ref[...]
    deg = lax.dot_general(cnt, jnp.ones((2, 1), jnp.float32),
                          (((0,), (0,)), ((), ()))) + 1.0
    valid = lax.broadcasted_iota(jnp.int32, (NP, 1), 0) < N_PROT
    dinv = jnp.where(valid, lax.rsqrt(deg), 0.0)
    dinv_ref[...] = dinv
    xw = jnp.dot(x_ref[...], w1_ref[...], preferred_element_type=jnp.float32)
    s1_ref[...] = jnp.zeros((NP, 32), jnp.float32)
    s1_ref[0:N_PROT, :] = xw * dinv[0:N_PROT, :]


def _tc_b(agg_ref, s1_ref, dinv_ref, b1_ref, w2_ref, s2_ref):
    dinv = dinv_ref[...]
    h1 = jax.nn.relu(dinv * (agg_ref[0] + agg_ref[1] + s1_ref[...])
                     + b1_ref[...])
    s2_ref[...] = jnp.dot(h1, w2_ref[...],
                          preferred_element_type=jnp.float32) * dinv


def _tc_c(agg_ref, s2_ref, dinv_ref, b2_ref, h2p_ref):
    dinv = dinv_ref[...]
    valid = lax.broadcasted_iota(jnp.int32, (NP, 1), 0) < N_PROT
    h2 = jnp.where(valid,
                   dinv * (agg_ref[0] + agg_ref[1] + s2_ref[...])
                   + b2_ref[...], 0.0)
    h2p_ref[...] = jnp.zeros((NT, 16), jnp.float32)
    h2p_ref[0:NP, :] = h2


def _tc_d(ssum_ref, cnt_ref, wh_ref, xd_ref, emb_ref, wf1_ref, bf1_ref,
          wf2_ref, bf2_ref, out_ref):
    ssum = ssum_ref[0][N_PROT:TOT, :] + ssum_ref[1][N_PROT:TOT, :]
    cnt = lax.dot_general(cnt_ref[...], jnp.ones((2, 1), jnp.float32),
                          (((0,), (0,)), ((), ())))
    cnt = lax.slice(cnt, (N_PROT, 0), (TOT, 1))
    aggr = ssum / jnp.maximum(cnt, 1.0)
    prot_out = jnp.dot(aggr, wh_ref[...], preferred_element_type=jnp.float32)
    xd = jnp.dot(xd_ref[...], emb_ref[...], preferred_element_type=jnp.float32)
    g = (jnp.dot(jax.nn.relu(xd), wf1_ref[0:48, :],
                 preferred_element_type=jnp.float32)
         + jnp.dot(jax.nn.relu(prot_out), wf1_ref[48:64, :],
                   preferred_element_type=jnp.float32)
         + bf1_ref[...])
    h = jax.nn.relu(g)
    f = jnp.dot(h, wf2_ref[...], preferred_element_type=jnp.float32) \
        + bf2_ref[...]
    out_ref[...] = jax.nn.sigmoid(f)


def _tc_call(fn, out_shapes):
    return pl.pallas_call(fn, out_shape=out_shapes)


def kernel(x_prot, pp_edge_index, dp_edge_index, dp_range_list, x_drug,
           W1, b1, W2, b2, Wh, embed, Wf1, bf1, Wf2, bf2):
    f32 = jnp.float32
    pp2 = jnp.pad(pp_edge_index.astype(jnp.int32),
                  ((0, 0), (0, EP_PAD - E_PP)), constant_values=NP - 1)
    src_pp = pp2[0].reshape(TOTB, B)
    col_pp = pp2[1].reshape(TOTB, B)
    src_dp = dp_edge_index[0].astype(jnp.int32).reshape(NC * NS * NBW_DP, B)
    dst_dp = dp_edge_index[1].astype(jnp.int32).reshape(NC * NS * NBW_DP, B)
    b1r, b2r = b1.reshape(1, -1), b2.reshape(1, -1)
    bf1r, bf2r = bf1.reshape(1, -1), bf2.reshape(1, -1)

    cnt_pp, cnt_dp = _sc_counts(col_pp, dst_dp)
    cnt_pp = cnt_pp.reshape(NC, NP)
    cnt_dp = cnt_dp.reshape(NC, NT)

    s1, dinv = _tc_call(_tc_a, [jax.ShapeDtypeStruct((NP, 32), f32),
                                jax.ShapeDtypeStruct((NP, 1), f32)])(
        x_prot, W1, cnt_pp)

    agg1 = _make_sc_aggregate(NP, 32, NB0, NB1)(s1, src_pp, col_pp)

    s2 = _tc_call(_tc_b, jax.ShapeDtypeStruct((NP, 16), f32))(
        agg1, s1, dinv, b1r, W2)

    agg2 = _make_sc_aggregate(NP, 16, NB0, NB1)(s2, src_pp, col_pp)

    h2p = _tc_call(_tc_c, jax.ShapeDtypeStruct((NT, 16), f32))(
        agg2, s2, dinv, b2r)

    ssum = _make_sc_aggregate(NT, 16, NBW_DP, NBW_DP)(h2p, src_dp, dst_dp)

    out = _tc_call(_tc_d, jax.ShapeDtypeStruct((N_DRUG, N_DRUG), f32))(
        ssum, cnt_dp, Wh, x_drug, embed, Wf1, bf1r, Wf2, bf2r)
    return out.reshape(-1)


# speedup vs baseline: 1.7661x; 1.4469x over previous
import functools

import jax
import jax.numpy as jnp
from jax import lax
from jax.experimental import pallas as pl
from jax.experimental.pallas import tpu as pltpu
from jax.experimental.pallas import tpu_sc as plsc

N_PROT = 10000
N_DRUG = 1024
TOT = N_PROT + N_DRUG
NP = 10240
NT = 11264
E_PP = 320000
E_DP = 32768
NC, NS, B = 2, 16, 128
NB0, NB1 = 80, 80
TOTB = NS * (NB0 + NB1)
EP_PAD = TOTB * B
NBW_CNT = TOTB // (NC * NS)
NBW_DP = 8

_MESH = plsc.VectorSubcoreMesh(core_axis_name="c", subcore_axis_name="s")


def _zero_fill(ref, nrows, d):
    def row(i, _):
        for k in range(d // 16):
            ref[i, pl.ds(16 * k, 16)] = jnp.zeros((16,), jnp.float32)
        return 0
    lax.fori_loop(0, nrows, row, 0)


def _make_sc_aggregate(nr, d, nbw0, nbw1):
    rows_pt = nr // NS
    nbw_max = max(nbw0, nbw1)

    @functools.partial(
        pl.kernel,
        out_type=jax.ShapeDtypeStruct((NC, nr, d), jnp.float32),
        mesh=_MESH,
        scratch_types=[
            pltpu.VMEM((nbw_max, B), jnp.int32),
            pltpu.VMEM((nbw_max, B), jnp.int32),
            [pltpu.VMEM((B, d), jnp.float32) for _ in range(4)],
            pltpu.VMEM((64, d), jnp.float32),
            pltpu.VMEM_SHARED((nr, d), jnp.float32),
            [pltpu.SemaphoreType.DMA for _ in range(4)],
            [pltpu.SemaphoreType.DMA for _ in range(4)],
            pltpu.SemaphoreType.DMA,
        ],
        compiler_params=pltpu.CompilerParams(use_tc_tiling_on_sc=False),
    )
    def agg(table_hbm, src_hbm, dst_hbm, out_hbm, sidx, didx, rows,
            zbuf, acc, gsem, ssem, zsem):
        c = lax.axis_index("c")
        s = lax.axis_index("s")
        _zero_fill(zbuf, 64, d)

        nz = rows_pt // 64
        def zacc(i, _):
            pltpu.async_copy(
                zbuf, acc.at[pl.ds(s * rows_pt + i * 64, 64)], zsem)
            return 0
        lax.fori_loop(0, nz, zacc, 0)

        def zdrain(i, _):
            pltpu.make_async_copy(
                zbuf, acc.at[pl.ds(s * rows_pt, 64)], zsem).wait()
            return 0
        lax.fori_loop(0, nz, zdrain, 0)
        plsc.subcore_barrier()

        if nbw0 == nbw1:
            nbw = nbw0
            base = (c * NS + s) * nbw0
            pltpu.sync_copy(src_hbm.at[pl.ds(base, nbw0)], sidx)
            pltpu.sync_copy(dst_hbm.at[pl.ds(base, nbw0)], didx)
        else:
            nbw = jnp.where(c == 0, nbw0, nbw1)

            @pl.when(c == 0)
            def _():
                pltpu.sync_copy(src_hbm.at[pl.ds(s * nbw0, nbw0)],
                                sidx.at[pl.ds(0, nbw0)])
                pltpu.sync_copy(dst_hbm.at[pl.ds(s * nbw0, nbw0)],
                                didx.at[pl.ds(0, nbw0)])

            @pl.when(c == 1)
            def _():
                pltpu.sync_copy(src_hbm.at[pl.ds(NS * nbw0 + s * nbw1, nbw1)],
                                sidx.at[pl.ds(0, nbw1)])
                pltpu.sync_copy(dst_hbm.at[pl.ds(NS * nbw0 + s * nbw1, nbw1)],
                                didx.at[pl.ds(0, nbw1)])

        def gather(j, k):
            return pltpu.make_async_copy(table_hbm.at[sidx.at[j]],
                                         rows[k], gsem[k])

        def sdrain(k):
            pltpu.make_async_copy(rows[k], acc.at[didx.at[0]],
                                  ssem[k]).wait()

        nq = nbw // 4
        gather(0, 0).start()
        gather(1, 1).start()
        gather(2, 2).start()

        def step(q, _):
            for k in range(4):
                j = 4 * q + k
                kp = (k + 3) % 4
                gather(j, k).wait()
                pltpu.async_copy(rows[k], acc.at[didx.at[j]], ssem[k],
                                 add=True)
                if k == 0:
                    @pl.when(q > 0)
                    def _():
                        sdrain(kp)
                    gather(j + 3, kp).start()
                else:
                    sdrain(kp)

                    @pl.when(q < nq - 1)
                    def _():
                        gather(j + 3, kp).start()
            return 0
        lax.fori_loop(0, nq, step, 0)
        sdrain(3)
        plsc.subcore_barrier()

        pltpu.sync_copy(acc.at[pl.ds(s * rows_pt, rows_pt)],
                        out_hbm.at[c, pl.ds(s * rows_pt, rows_pt)])

    return agg


@functools.partial(
    pl.kernel,
    out_type=[jax.ShapeDtypeStruct((NC * NP,), jnp.float32),
              jax.ShapeDtypeStruct((NC * NT,), jnp.float32)],
    mesh=_MESH,
    scratch_types=[
        pltpu.VMEM((NBW_CNT, B), jnp.int32),
        pltpu.VMEM((NBW_DP, B), jnp.int32),
        pltpu.VMEM((B,), jnp.float32),
        pltpu.VMEM((NT // NS,), jnp.float32),
        pltpu.VMEM_SHARED((NP,), jnp.float32),
        pltpu.VMEM_SHARED((NT,), jnp.float32),
        pltpu.SemaphoreType.DMA,
    ],
    compiler_params=pltpu.CompilerParams(use_tc_tiling_on_sc=False),
)
def _sc_counts(col_hbm, dpd_hbm, outp_hbm, outt_hbm,
               cidx, didx, ones, zbuf, accp, acct, csem):
    c = lax.axis_index("c")
    s = lax.axis_index("s")
    pp_pt = NP // NS
    tt_pt = NT // NS
    def fill(i, _):
        zbuf[pl.ds(i * 16, 16)] = jnp.zeros((16,), jnp.float32)
        return 0
    lax.fori_loop(0, tt_pt // 16, fill, 0)
    for k in range(B // 16):
        ones[pl.ds(16 * k, 16)] = jnp.ones((16,), jnp.float32)
    pltpu.sync_copy(zbuf.at[pl.ds(0, pp_pt)], accp.at[pl.ds(s * pp_pt, pp_pt)])
    pltpu.sync_copy(zbuf, acct.at[pl.ds(s * tt_pt, tt_pt)])
    plsc.subcore_barrier()

    w = c * NS + s
    pltpu.sync_copy(col_hbm.at[pl.ds(w * NBW_CNT, NBW_CNT)], cidx)
    pltpu.sync_copy(dpd_hbm.at[pl.ds(w * NBW_DP, NBW_DP)], didx)

    def cbatch(j, _):
        pltpu.sync_copy(ones, accp.at[cidx.at[j]], add=True)
        return 0
    lax.fori_loop(0, NBW_CNT, cbatch, 0)

    def dbatch(j, _):
        pltpu.sync_copy(ones, acct.at[didx.at[j]], add=True)
        return 0
    lax.fori_loop(0, NBW_DP, dbatch, 0)
    plsc.subcore_barrier()

    pltpu.sync_copy(accp.at[pl.ds(s * pp_pt, pp_pt)], zbuf.at[pl.ds(0, pp_pt)])
    pltpu.sync_copy(zbuf.at[pl.ds(0, pp_pt)],
                    outp_hbm.at[pl.ds(c * NP + s * pp_pt, pp_pt)])
    pltpu.sync_copy(acct.at[pl.ds(s * tt_pt, tt_pt)], zbuf)
    pltpu.sync_copy(zbuf, outt_hbm.at[pl.ds(c * NT + s * tt_pt, tt_pt)])


def _tc_a(x_ref, w1_ref, cnt_ref, s1_ref, dinv_ref):
    cnt = cnt_ref[...]
    deg = lax.dot_general(cnt, jnp.ones((2, 1), jnp.float32),
                          (((0,), (0,)), ((), ()))) + 1.0
    valid = lax.broadcasted_iota(jnp.int32, (NP, 1), 0) < N_PROT
    dinv = jnp.where(valid, lax.rsqrt(deg), 0.0)
    dinv_ref[...] = dinv
    xw = jnp.dot(x_ref[...], w1_ref[...], preferred_element_type=jnp.float32)
    s1_ref[...] = jnp.zeros((NP, 32), jnp.float32)
    s1_ref[0:N_PROT, :] = xw * dinv[0:N_PROT, :]


def _tc_b(agg_ref, s1_ref, dinv_ref, b1_ref, w2_ref, s2_ref):
    dinv = dinv_ref[...]
    h1 = jax.nn.relu(dinv * (agg_ref[0] + agg_ref[1] + s1_ref[...])
                     + b1_ref[...])
    s2_ref[...] = jnp.dot(h1, w2_ref[...],
                          preferred_element_type=jnp.float32) * dinv


def _tc_c(agg_ref, s2_ref, dinv_ref, b2_ref, h2p_ref):
    dinv = dinv_ref[...]
    valid = lax.broadcasted_iota(jnp.int32, (NP, 1), 0) < N_PROT
    h2 = jnp.where(valid,
                   dinv * (agg_ref[0] + agg_ref[1] + s2_ref[...])
                   + b2_ref[...], 0.0)
    h2p_ref[...] = jnp.zeros((NT, 16), jnp.float32)
    h2p_ref[0:NP, :] = h2


def _tc_d(ssum_ref, cnt_ref, wh_ref, xd_ref, emb_ref, wf1_ref, bf1_ref,
          wf2_ref, bf2_ref, out_ref):
    ssum = ssum_ref[0][N_PROT:TOT, :] + ssum_ref[1][N_PROT:TOT, :]
    cnt = lax.dot_general(cnt_ref[...], jnp.ones((2, 1), jnp.float32),
                          (((0,), (0,)), ((), ())))
    cnt = lax.slice(cnt, (N_PROT, 0), (TOT, 1))
    aggr = ssum / jnp.maximum(cnt, 1.0)
    prot_out = jnp.dot(aggr, wh_ref[...], preferred_element_type=jnp.float32)
    xd = jnp.dot(xd_ref[...], emb_ref[...], preferred_element_type=jnp.float32)
    g = (jnp.dot(jax.nn.relu(xd), wf1_ref[0:48, :],
                 preferred_element_type=jnp.float32)
         + jnp.dot(jax.nn.relu(prot_out), wf1_ref[48:64, :],
                   preferred_element_type=jnp.float32)
         + bf1_ref[...])
    h = jax.nn.relu(g)
    f = jnp.dot(h, wf2_ref[...], preferred_element_type=jnp.float32) \
        + bf2_ref[...]
    out_ref[...] = jax.nn.sigmoid(f)


def _tc_call(fn, out_shapes):
    return pl.pallas_call(fn, out_shape=out_shapes)


def kernel(x_prot, pp_edge_index, dp_edge_index, dp_range_list, x_drug,
           W1, b1, W2, b2, Wh, embed, Wf1, bf1, Wf2, bf2):
    f32 = jnp.float32
    pads = (jnp.arange(EP_PAD - E_PP, dtype=jnp.int32) % (NP - N_PROT)
            + N_PROT)
    pp2 = jnp.concatenate(
        [pp_edge_index.astype(jnp.int32), jnp.stack([pads, pads])], axis=1)
    src_pp = pp2[0].reshape(TOTB, B)
    col_pp = pp2[1].reshape(TOTB, B)
    src_dp = dp_edge_index[0].astype(jnp.int32).reshape(NC * NS * NBW_DP, B)
    dst_dp = dp_edge_index[1].astype(jnp.int32).reshape(NC * NS * NBW_DP, B)
    b1r, b2r = b1.reshape(1, -1), b2.reshape(1, -1)
    bf1r, bf2r = bf1.reshape(1, -1), bf2.reshape(1, -1)

    cnt_pp, cnt_dp = _sc_counts(col_pp, dst_dp)
    cnt_pp = cnt_pp.reshape(NC, NP)
    cnt_dp = cnt_dp.reshape(NC, NT)

    s1, dinv = _tc_call(_tc_a, [jax.ShapeDtypeStruct((NP, 32), f32),
                                jax.ShapeDtypeStruct((NP, 1), f32)])(
        x_prot, W1, cnt_pp)

    agg1 = _make_sc_aggregate(NP, 32, NB0, NB1)(s1, src_pp, col_pp)

    s2 = _tc_call(_tc_b, jax.ShapeDtypeStruct((NP, 16), f32))(
        agg1, s1, dinv, b1r, W2)

    agg2 = _make_sc_aggregate(NP, 16, NB0, NB1)(s2, src_pp, col_pp)

    h2p = _tc_call(_tc_c, jax.ShapeDtypeStruct((NT, 16), f32))(
        agg2, s2, dinv, b2r)

    ssum = _make_sc_aggregate(NT, 16, NBW_DP, NBW_DP)(h2p, src_dp, dst_dp)

    out = _tc_call(_tc_d, jax.ShapeDtypeStruct((N_DRUG, N_DRUG), f32))(
        ssum, cnt_dp, Wh, x_drug, embed, Wf1, bf1r, Wf2, bf2r)
    return out.reshape(-1)


# speedup vs baseline: 1.8149x; 1.0276x over previous
import functools

import jax
import jax.numpy as jnp
from jax import lax
from jax.experimental import pallas as pl
from jax.experimental.pallas import tpu as pltpu
from jax.experimental.pallas import tpu_sc as plsc

N_PROT = 10000
N_DRUG = 1024
TOT = N_PROT + N_DRUG
NP = 10240
NT = 11264
E_PP = 320000
E_DP = 32768
NC, NS, B = 2, 16, 128
NB0, NB1 = 80, 80
TOTB = NS * (NB0 + NB1)
EP_PAD = TOTB * B
NBW_CNT = TOTB // (NC * NS)
NBW_DP = 8

_MESH = plsc.VectorSubcoreMesh(core_axis_name="c", subcore_axis_name="s")


def _zero_fill(ref, nrows, d):
    def row(i, _):
        for k in range(d // 16):
            ref[i, pl.ds(16 * k, 16)] = jnp.zeros((16,), jnp.float32)
        return 0
    lax.fori_loop(0, nrows, row, 0)


def _make_sc_aggregate(nr, d, nbw0, nbw1):
    rows_pt = nr // NS
    nbw_max = max(nbw0, nbw1)

    @functools.partial(
        pl.kernel,
        out_type=jax.ShapeDtypeStruct((NC, nr, d), jnp.float32),
        mesh=_MESH,
        scratch_types=[
            pltpu.VMEM((nbw_max, B), jnp.int32),
            pltpu.VMEM((nbw_max, B), jnp.int32),
            [pltpu.VMEM((B, d), jnp.float32) for _ in range(4)],
            pltpu.VMEM((64, d), jnp.float32),
            pltpu.VMEM_SHARED((nr, d), jnp.float32),
            [pltpu.SemaphoreType.DMA for _ in range(4)],
            [pltpu.SemaphoreType.DMA for _ in range(4)],
            pltpu.SemaphoreType.DMA,
        ],
        compiler_params=pltpu.CompilerParams(use_tc_tiling_on_sc=False),
    )
    def agg(table_hbm, src_hbm, dst_hbm, out_hbm, sidx, didx, rows,
            zbuf, acc, gsem, ssem, zsem):
        c = lax.axis_index("c")
        s = lax.axis_index("s")
        _zero_fill(zbuf, 64, d)

        nz = rows_pt // 64
        def zacc(i, _):
            pltpu.async_copy(
                zbuf, acc.at[pl.ds(s * rows_pt + i * 64, 64)], zsem)
            return 0
        lax.fori_loop(0, nz, zacc, 0)

        def zdrain(i, _):
            pltpu.make_async_copy(
                zbuf, acc.at[pl.ds(s * rows_pt, 64)], zsem).wait()
            return 0
        lax.fori_loop(0, nz, zdrain, 0)
        plsc.subcore_barrier()

        if nbw0 == nbw1:
            nbw = nbw0
            base = (c * NS + s) * nbw0
            pltpu.sync_copy(src_hbm.at[pl.ds(base, nbw0)], sidx)
            pltpu.sync_copy(dst_hbm.at[pl.ds(base, nbw0)], didx)
        else:
            nbw = jnp.where(c == 0, nbw0, nbw1)

            @pl.when(c == 0)
            def _():
                pltpu.sync_copy(src_hbm.at[pl.ds(s * nbw0, nbw0)],
                                sidx.at[pl.ds(0, nbw0)])
                pltpu.sync_copy(dst_hbm.at[pl.ds(s * nbw0, nbw0)],
                                didx.at[pl.ds(0, nbw0)])

            @pl.when(c == 1)
            def _():
                pltpu.sync_copy(src_hbm.at[pl.ds(NS * nbw0 + s * nbw1, nbw1)],
                                sidx.at[pl.ds(0, nbw1)])
                pltpu.sync_copy(dst_hbm.at[pl.ds(NS * nbw0 + s * nbw1, nbw1)],
                                didx.at[pl.ds(0, nbw1)])

        def gather(j, k):
            return pltpu.make_async_copy(table_hbm.at[sidx.at[j]],
                                         rows[k], gsem[k])

        def sdrain(k):
            pltpu.make_async_copy(rows[k], acc.at[didx.at[0]],
                                  ssem[k]).wait()

        nq = nbw // 4
        gather(0, 0).start()
        gather(1, 1).start()
        gather(2, 2).start()

        def step(q, _):
            for k in range(4):
                j = 4 * q + k
                kp = (k + 3) % 4
                gather(j, k).wait()
                pltpu.async_copy(rows[k], acc.at[didx.at[j]], ssem[k],
                                 add=True)
                if k == 0:
                    @pl.when(q > 0)
                    def _():
                        sdrain(kp)
                    gather(j + 3, kp).start()
                else:
                    sdrain(kp)

                    @pl.when(q < nq - 1)
                    def _():
                        gather(j + 3, kp).start()
            return 0
        lax.fori_loop(0, nq, step, 0)
        sdrain(3)
        plsc.subcore_barrier()

        pltpu.sync_copy(acc.at[pl.ds(s * rows_pt, rows_pt)],
                        out_hbm.at[c, pl.ds(s * rows_pt, rows_pt)])

    return agg


@functools.partial(
    pl.kernel,
    out_type=[jax.ShapeDtypeStruct((NC * NP,), jnp.float32),
              jax.ShapeDtypeStruct((NC * NT,), jnp.float32)],
    mesh=_MESH,
    scratch_types=[
        pltpu.VMEM((NBW_CNT, B), jnp.int32),
        pltpu.VMEM((NBW_DP, B), jnp.int32),
        pltpu.VMEM((B,), jnp.float32),
        pltpu.VMEM((NT // NS,), jnp.float32),
        pltpu.VMEM_SHARED((NP,), jnp.float32),
        pltpu.VMEM_SHARED((NT,), jnp.float32),
        pltpu.SemaphoreType.DMA,
    ],
    compiler_params=pltpu.CompilerParams(use_tc_tiling_on_sc=False),
)
def _sc_counts(col_hbm, dpd_hbm, outp_hbm, outt_hbm,
               cidx, didx, ones, zbuf, accp, acct, csem):
    c = lax.axis_index("c")
    s = lax.axis_index("s")
    pp_pt = NP // NS
    tt_pt = NT // NS
    def fill(i, _):
        zbuf[pl.ds(i * 16, 16)] = jnp.zeros((16,), jnp.float32)
        return 0
    lax.fori_loop(0, tt_pt // 16, fill, 0)
    for k in range(B // 16):
        ones[pl.ds(16 * k, 16)] = jnp.ones((16,), jnp.float32)
    pltpu.sync_copy(zbuf.at[pl.ds(0, pp_pt)], accp.at[pl.ds(s * pp_pt, pp_pt)])
    pltpu.sync_copy(zbuf, acct.at[pl.ds(s * tt_pt, tt_pt)])
    plsc.subcore_barrier()

    w = c * NS + s
    pltpu.sync_copy(col_hbm.at[pl.ds(w * NBW_CNT, NBW_CNT)], cidx)
    pltpu.sync_copy(dpd_hbm.at[pl.ds(w * NBW_DP, NBW_DP)], didx)

    def cchunk(q, _):
        for k in range(4):
            pltpu.async_copy(ones, accp.at[cidx.at[4 * q + k]], csem,
                             add=True)
        for k in range(4):
            pltpu.make_async_copy(ones, accp.at[cidx.at[0]], csem).wait()
        return 0
    lax.fori_loop(0, NBW_CNT // 4, cchunk, 0)

    for k in range(NBW_DP):
        pltpu.async_copy(ones, acct.at[didx.at[k]], csem, add=True)
    for k in range(NBW_DP):
        pltpu.make_async_copy(ones, acct.at[didx.at[0]], csem).wait()
    plsc.subcore_barrier()

    pltpu.sync_copy(accp.at[pl.ds(s * pp_pt, pp_pt)], zbuf.at[pl.ds(0, pp_pt)])
    pltpu.sync_copy(zbuf.at[pl.ds(0, pp_pt)],
                    outp_hbm.at[pl.ds(c * NP + s * pp_pt, pp_pt)])
    pltpu.sync_copy(acct.at[pl.ds(s * tt_pt, tt_pt)], zbuf)
    pltpu.sync_copy(zbuf, outt_hbm.at[pl.ds(c * NT + s * tt_pt, tt_pt)])


def _tc_a(x_ref, w1_ref, cnt_ref, s1_ref, dinv_ref):
    cnt = cnt_ref[...]
    deg = lax.dot_general(cnt, jnp.ones((2, 1), jnp.float32),
                          (((0,), (0,)), ((), ()))) + 1.0
    valid = lax.broadcasted_iota(jnp.int32, (NP, 1), 0) < N_PROT
    dinv = jnp.where(valid, lax.rsqrt(deg), 0.0)
    dinv_ref[...] = dinv
    xw = jnp.dot(x_ref[...], w1_ref[...], preferred_element_type=jnp.float32)
    s1_ref[...] = jnp.zeros((NP, 32), jnp.float32)
    s1_ref[0:N_PROT, :] = xw * dinv[0:N_PROT, :]


def _tc_b(agg_ref, s1_ref, dinv_ref, b1_ref, w2_ref, s2_ref):
    dinv = dinv_ref[...]
    h1 = jax.nn.relu(dinv * (agg_ref[0] + agg_ref[1] + s1_ref[...])
                     + b1_ref[...])
    s2_ref[...] = jnp.dot(h1, w2_ref[...],
                          preferred_element_type=jnp.float32) * dinv


def _tc_c(agg_ref, s2_ref, dinv_ref, b2_ref, h2p_ref):
    dinv = dinv_ref[...]
    valid = lax.broadcasted_iota(jnp.int32, (NP, 1), 0) < N_PROT
    h2 = jnp.where(valid,
                   dinv * (agg_ref[0] + agg_ref[1] + s2_ref[...])
                   + b2_ref[...], 0.0)
    h2p_ref[...] = jnp.zeros((NT, 16), jnp.float32)
    h2p_ref[0:NP, :] = h2


def _tc_xd(xd_ref, emb_ref, out_ref):
    out_ref[...] = jnp.dot(xd_ref[...], emb_ref[...],
                           preferred_element_type=jnp.float32)


def _tc_d(ssum_ref, cnt_ref, wh_ref, xd_ref, wf1_ref, bf1_ref,
          wf2_ref, bf2_ref, out_ref):
    ssum = ssum_ref[0][N_PROT:TOT, :] + ssum_ref[1][N_PROT:TOT, :]
    cnt = lax.dot_general(cnt_ref[...], jnp.ones((2, 1), jnp.float32),
                          (((0,), (0,)), ((), ())))
    cnt = lax.slice(cnt, (N_PROT, 0), (TOT, 1))
    aggr = ssum / jnp.maximum(cnt, 1.0)
    prot_out = jnp.dot(aggr, wh_ref[...], preferred_element_type=jnp.float32)
    xd = xd_ref[...]
    g = (jnp.dot(jax.nn.relu(xd), wf1_ref[0:48, :],
                 preferred_element_type=jnp.float32)
         + jnp.dot(jax.nn.relu(prot_out), wf1_ref[48:64, :],
                   preferred_element_type=jnp.float32)
         + bf1_ref[...])
    h = jax.nn.relu(g)
    f = jnp.dot(h, wf2_ref[...], preferred_element_type=jnp.float32) \
        + bf2_ref[...]
    out_ref[...] = jax.nn.sigmoid(f)


def _tc_call(fn, out_shapes):
    return pl.pallas_call(fn, out_shape=out_shapes)


def kernel(x_prot, pp_edge_index, dp_edge_index, dp_range_list, x_drug,
           W1, b1, W2, b2, Wh, embed, Wf1, bf1, Wf2, bf2):
    f32 = jnp.float32
    pads = (jnp.arange(EP_PAD - E_PP, dtype=jnp.int32) % (NP - N_PROT)
            + N_PROT)
    pp2 = jnp.concatenate(
        [pp_edge_index.astype(jnp.int32), jnp.stack([pads, pads])], axis=1)
    src_pp = pp2[0].reshape(TOTB, B)
    col_pp = pp2[1].reshape(TOTB, B)
    src_dp = dp_edge_index[0].astype(jnp.int32).reshape(NC * NS * NBW_DP, B)
    dst_dp = dp_edge_index[1].astype(jnp.int32).reshape(NC * NS * NBW_DP, B)
    b1r, b2r = b1.reshape(1, -1), b2.reshape(1, -1)
    bf1r, bf2r = bf1.reshape(1, -1), bf2.reshape(1, -1)

    xd = _tc_call(_tc_xd, jax.ShapeDtypeStruct((N_DRUG, 48), f32))(
        x_drug, embed)

    cnt_pp, cnt_dp = _sc_counts(col_pp, dst_dp)
    cnt_pp = cnt_pp.reshape(NC, NP)
    cnt_dp = cnt_dp.reshape(NC, NT)

    s1, dinv = _tc_call(_tc_a, [jax.ShapeDtypeStruct((NP, 32), f32),
                                jax.ShapeDtypeStruct((NP, 1), f32)])(
        x_prot, W1, cnt_pp)

    agg1 = _make_sc_aggregate(NP, 32, NB0, NB1)(s1, src_pp, col_pp)

    s2 = _tc_call(_tc_b, jax.ShapeDtypeStruct((NP, 16), f32))(
        agg1, s1, dinv, b1r, W2)

    agg2 = _make_sc_aggregate(NP, 16, NB0, NB1)(s2, src_pp, col_pp)

    h2p = _tc_call(_tc_c, jax.ShapeDtypeStruct((NT, 16), f32))(
        agg2, s2, dinv, b2r)

    ssum = _make_sc_aggregate(NT, 16, NBW_DP, NBW_DP)(h2p, src_dp, dst_dp)

    out = _tc_call(_tc_d, jax.ShapeDtypeStruct((N_DRUG, N_DRUG), f32))(
        ssum, cnt_dp, Wh, xd, Wf1, bf1r, Wf2, bf2r)
    return out.reshape(-1)
